# baseline (speedup 1.0000x reference)
import functools

import jax
import jax.numpy as jnp
from jax import lax
from jax.experimental import pallas as pl
from jax.experimental.pallas import tpu as pltpu
from jax.experimental.pallas import tpu_sc as plsc

N_NODES = 10000
N_EDGES = 320000
EDGE_VOCAB = 10000
DIM = 128
L = 16
NV = DIM // L
NC, NS = 2, 16
NW = NC * NS
EPS = 1e-5

NODE_PW = N_NODES // NW
NODE_TAIL = N_NODES - NODE_PW * NW
NODE_CH = 104
NODE_NCH = NODE_PW // NODE_CH
EDGE_PW = N_EDGES // NW
EDGE_CH = 80
EDGE_NCH = EDGE_PW // EDGE_CH
NBUF = 5
DPRE_N = 2

VOC_PW = 624
VOC_TAIL = EDGE_VOCAB - VOC_PW * NS
VOC_CH = 208
VOC_NCH = VOC_PW // VOC_CH
HZERO = 2000

_mesh = plsc.VectorSubcoreMesh(core_axis_name="c", subcore_axis_name="s")

_GATHER_DNUMS = lax.GatherDimensionNumbers(
    offset_dims=(), collapsed_slice_dims=(0,), start_index_map=(0,))


def _bcast(v, i):
    idx = jnp.full((L, 1), i, dtype=jnp.int32)
    return lax.gather(v, idx, _GATHER_DNUMS, (1,),
                      mode=lax.GatherScatterMode.PROMISE_IN_BOUNDS)


def _rsqrt(x):
    i = lax.bitcast_convert_type(x, jnp.int32)
    y = lax.bitcast_convert_type(
        jnp.int32(0x5F3759DF) - lax.shift_right_logical(i, 1), jnp.float32)
    for _ in range(3):
        y = y * (1.5 - 0.5 * x * y * y)
    return y


def _accum_rows(rows_ref, n, carry):
    def body(r, c):
        sums, sqs = c
        new_s, new_q = [], []
        for j in range(NV):
            v = rows_ref[r, pl.ds(L * j, L)]
            new_s.append(sums[j] + v)
            new_q.append(sqs[j] + v * v)
        return tuple(new_s), tuple(new_q)
    return lax.fori_loop(0, n, body, carry)


def _store_carry(part_ref, carry):
    sums, sqs = carry
    for j in range(NV):
        part_ref[0, pl.ds(L * j, L)] = sums[j]
        part_ref[1, pl.ds(L * j, L)] = sqs[j]


def _waccum_rows(rows_ref, cnt_ref, cnt_base, ngroups, carry):
    def body(g, c):
        sums, sqs = c
        cv = cnt_ref[pl.ds(cnt_base + L * g, L)]
        new_s, new_q = list(sums), list(sqs)
        for i in range(L):
            w = _bcast(cv, i)
            for j in range(NV):
                v = rows_ref[L * g + i, pl.ds(L * j, L)]
                wv = w * v
                new_s[j] = new_s[j] + wv
                new_q[j] = new_q[j] + wv * v
        return tuple(new_s), tuple(new_q)
    return lax.fori_loop(0, ngroups, body, carry)


_STATS_SCRATCH = [
    pltpu.VMEM((EDGE_PW,), jnp.int32),
    pltpu.VMEM((NODE_PW + NODE_TAIL,), jnp.int32),
    pltpu.VMEM((NODE_NCH, NODE_CH, DIM), jnp.float32),
    pltpu.VMEM((VOC_CH, DIM), jnp.float32),
    pltpu.VMEM((2, DIM), jnp.float32),
    pltpu.VMEM((EDGE_PW,), jnp.float32),
    pltpu.VMEM((HZERO,), jnp.float32),
    pltpu.VMEM((VOC_PW + VOC_TAIL,), jnp.float32),
    pltpu.VMEM_SHARED((EDGE_VOCAB,), jnp.float32),
] + [pltpu.SemaphoreType.DMA] * NODE_NCH


@functools.partial(
    pl.kernel,
    out_type=[
        jax.ShapeDtypeStruct((NW, 2, DIM), jnp.float32),
        jax.ShapeDtypeStruct((NW, 2, DIM), jnp.float32),
    ],
    mesh=_mesh,
    scratch_types=_STATS_SCRATCH,
)
def _stats(x_hbm, ea_hbm, ntab_hbm, etab_hbm, npart_hbm, epart_hbm,
           idx_v, nidx_v, nring_v, vrows_v, part_v, ones_v, zero_v, cnt_v,
           hist_sp, *nsem):
    wid = lax.axis_index("s") * NC + lax.axis_index("c")
    sid = lax.axis_index("s")
    z = jnp.zeros((L,), jnp.float32)
    zero_carry = ((z,) * NV, (z,) * NV)

    nbase = wid * NODE_PW
    pltpu.sync_copy(x_hbm.at[pl.ds(nbase, NODE_PW)], nidx_v.at[pl.ds(0, NODE_PW)])

    @pl.when(wid == NW - 1)
    def _node_tail_idx():
        pltpu.sync_copy(x_hbm.at[pl.ds(NODE_PW * NW, NODE_TAIL)],
                        nidx_v.at[pl.ds(NODE_PW, NODE_TAIL)])

    for k in range(NODE_NCH):
        pltpu.async_copy(
            ntab_hbm.at[nidx_v.at[pl.ds(k * NODE_CH, NODE_CH)]],
            nring_v.at[k], nsem[k])

    ebase = wid * EDGE_PW
    pltpu.sync_copy(ea_hbm.at[pl.ds(ebase, EDGE_PW)], idx_v)

    carry = zero_carry
    for k in range(NODE_NCH):
        pltpu.make_async_copy(
            ntab_hbm.at[nidx_v.at[pl.ds(0, NODE_CH)]],
            nring_v.at[k], nsem[k]).wait()
        carry = _accum_rows(nring_v.at[k], NODE_CH, carry)
    _store_carry(part_v, carry)

    @pl.when(wid == NW - 1)
    def _node_tail():
        pltpu.async_copy(
            ntab_hbm.at[nidx_v.at[pl.ds(NODE_PW, NODE_TAIL)]],
            nring_v.at[0].at[pl.ds(0, NODE_TAIL)], nsem[0]).wait()
        tsum, tsq = _accum_rows(nring_v.at[0], NODE_TAIL,
                                ((z,) * NV, (z,) * NV))
        for j in range(NV):
            part_v[0, pl.ds(L * j, L)] = part_v[0, pl.ds(L * j, L)] + tsum[j]
            part_v[1, pl.ds(L * j, L)] = part_v[1, pl.ds(L * j, L)] + tsq[j]

    pltpu.sync_copy(part_v, npart_hbm.at[wid])

    one16 = jnp.ones((L,), jnp.float32)

    def fill_ones(i, _):
        ones_v[pl.ds(L * i, L)] = one16
        return 0
    lax.fori_loop(0, EDGE_PW // L, fill_ones, 0)

    @pl.when(sid < EDGE_VOCAB // HZERO)
    def _zero_hist():
        def fill_z(i, _):
            zero_v[pl.ds(L * i, L)] = z
            return 0
        lax.fori_loop(0, HZERO // L, fill_z, 0)
        pltpu.sync_copy(zero_v, hist_sp.at[pl.ds(sid * HZERO, HZERO)])

    plsc.subcore_barrier()
    pltpu.sync_copy(ones_v, hist_sp.at[idx_v], add=True)
    plsc.subcore_barrier()

    vbase = sid * VOC_PW
    pltpu.sync_copy(hist_sp.at[pl.ds(vbase, VOC_PW)], cnt_v.at[pl.ds(0, VOC_PW)])

    carry = zero_carry
    for c in range(VOC_NCH):
        pltpu.sync_copy(etab_hbm.at[pl.ds(vbase + c * VOC_CH, VOC_CH)], vrows_v)
        carry = _waccum_rows(vrows_v, cnt_v, c * VOC_CH, VOC_CH // L, carry)
    _store_carry(part_v, carry)

    @pl.when(sid == NS - 1)
    def _voc_tail():
        tbase = VOC_PW * NS
        pltpu.sync_copy(hist_sp.at[pl.ds(tbase, VOC_TAIL)],
                        cnt_v.at[pl.ds(VOC_PW, VOC_TAIL)])
        pltpu.sync_copy(etab_hbm.at[pl.ds(tbase, VOC_TAIL)],
                        vrows_v.at[pl.ds(0, VOC_TAIL)])
        tsum, tsq = _waccum_rows(vrows_v, cnt_v, VOC_PW, VOC_TAIL // L,
                                 ((z,) * NV, (z,) * NV))
        for j in range(NV):
            part_v[0, pl.ds(L * j, L)] = part_v[0, pl.ds(L * j, L)] + tsum[j]
            part_v[1, pl.ds(L * j, L)] = part_v[1, pl.ds(L * j, L)] + tsq[j]

    pltpu.sync_copy(part_v, epart_hbm.at[wid])


_NORM_SCRATCH = [
    pltpu.VMEM((EDGE_PW,), jnp.int32),
    pltpu.VMEM((NODE_CH, DIM), jnp.float32),
    pltpu.VMEM((NBUF, EDGE_CH, DIM), jnp.float32),
    pltpu.VMEM((NW, 2, DIM), jnp.float32),
    pltpu.VMEM((4, DIM), jnp.float32),
    pltpu.SemaphoreType.DMA,
] + [pltpu.SemaphoreType.DMA] * NBUF \
  + [pltpu.SemaphoreType.DMA] * NBUF


@functools.partial(
    pl.kernel,
    out_type=[
        jax.ShapeDtypeStruct((N_NODES, DIM), jnp.float32),
        jax.ShapeDtypeStruct((N_EDGES, DIM), jnp.float32),
    ],
    mesh=_mesh,
    scratch_types=_NORM_SCRATCH,
)
def _normalize(x_hbm, ea_hbm, ntab_hbm, etab_hbm, npart_hbm, epart_hbm,
               gb_hbm, h_hbm, e_hbm, idx_v, nrows_v, ring_v, part_v, gb_v,
               nsem, *sems):
    gsem, wsem = sems[:NBUF], sems[NBUF:]
    wid = lax.axis_index("s") * NC + lax.axis_index("c")
    z = jnp.zeros((L,), jnp.float32)

    def transform(rows_ref, n, a, b):
        def body(r, _):
            for j in range(NV):
                rows_ref[r, pl.ds(L * j, L)] = (
                    rows_ref[r, pl.ds(L * j, L)] * a[j] + b[j])
            return 0
        lax.fori_loop(0, n, body, 0)

    def fold(part_hbm, n, grow, brow):
        pltpu.sync_copy(part_hbm, part_v)

        def red(w, c):
            s, q = c
            ns = tuple(s[j] + part_v[w, 0, pl.ds(L * j, L)] for j in range(NV))
            nq = tuple(q[j] + part_v[w, 1, pl.ds(L * j, L)] for j in range(NV))
            return ns, nq

        s, q = lax.fori_loop(0, NW, red, ((z,) * NV, (z,) * NV))
        inv_n = jnp.float32(1.0 / n)
        a, b = [], []
        for j in range(NV):
            mean = s[j] * inv_n
            var = q[j] * inv_n - mean * mean
            aj = gb_v[grow, pl.ds(L * j, L)] * _rsqrt(var + EPS)
            a.append(aj)
            b.append(gb_v[brow, pl.ds(L * j, L)] - mean * aj)
        return a, b

    nbase = wid * NODE_PW
    pltpu.sync_copy(x_hbm.at[pl.ds(nbase, NODE_PW)], idx_v.at[pl.ds(0, NODE_PW)])
    pltpu.sync_copy(gb_hbm, gb_v)
    na, nb = fold(npart_hbm, N_NODES, 0, 1)
    for k in range(NODE_NCH):
        pltpu.async_copy(
            ntab_hbm.at[idx_v.at[pl.ds(k * NODE_CH, NODE_CH)]],
            nrows_v, nsem).wait()
        transform(nrows_v, NODE_CH, na, nb)
        pltpu.sync_copy(nrows_v,
                        h_hbm.at[pl.ds(nbase + k * NODE_CH, NODE_CH)])

    @pl.when(wid == NW - 1)
    def _node_tail():
        pltpu.sync_copy(x_hbm.at[pl.ds(NODE_PW * NW, NODE_TAIL)],
                        idx_v.at[pl.ds(0, NODE_TAIL)])
        pltpu.async_copy(
            ntab_hbm.at[idx_v.at[pl.ds(0, NODE_TAIL)]],
            nrows_v.at[pl.ds(0, NODE_TAIL)], nsem).wait()
        transform(nrows_v, NODE_TAIL, na, nb)
        pltpu.sync_copy(nrows_v.at[pl.ds(0, NODE_TAIL)],
                        h_hbm.at[pl.ds(NODE_PW * NW, NODE_TAIL)])

    ebase = wid * EDGE_PW
    pltpu.sync_copy(ea_hbm.at[pl.ds(ebase, EDGE_PW)], idx_v)
    ea, eb = fold(epart_hbm, N_EDGES, 2, 3)

    def gather(chunk, buf, sem):
        pltpu.async_copy(
            etab_hbm.at[idx_v.at[pl.ds(chunk * EDGE_CH, EDGE_CH)]],
            ring_v.at[buf], sem)

    def gwait(buf, sem):
        pltpu.make_async_copy(
            etab_hbm.at[idx_v.at[pl.ds(0, EDGE_CH)]],
            ring_v.at[buf], sem).wait()

    def wdrain(buf, sem):
        pltpu.make_async_copy(
            ring_v.at[buf], e_hbm.at[pl.ds(ebase, EDGE_CH)], sem).wait()

    for b in range(DPRE_N):
        gather(b, b, gsem[b])

    def group(g, _):
        for b in range(NBUF):
            k = g * NBUF + b
            gwait(b, gsem[b])
            transform(ring_v.at[b], EDGE_CH, ea, eb)
            pltpu.async_copy(
                ring_v.at[b],
                e_hbm.at[pl.ds(ebase + k * EDGE_CH, EDGE_CH)], wsem[b])
            bn = (b + DPRE_N) % NBUF

            @pl.when(k + DPRE_N < EDGE_NCH)
            def _rearm():
                @pl.when(k + DPRE_N >= NBUF)
                def _drain_prev():
                    wdrain(bn, wsem[bn])
                gather(k + DPRE_N, bn, gsem[bn])
        return 0

    lax.fori_loop(0, EDGE_NCH // NBUF, group, 0)
    for b in range(NBUF):
        wdrain(b, wsem[b])


def kernel(x, edge_index, edge_attr, node_table, edge_table,
           node_gamma, node_beta, edge_gamma, edge_beta):
    del edge_index
    x = x.astype(jnp.int32)
    edge_attr = edge_attr.astype(jnp.int32)
    gb = jnp.stack([node_gamma, node_beta, edge_gamma, edge_beta])
    npart, epart = _stats(x, edge_attr, node_table, edge_table)
    h, e = _normalize(x, edge_attr, node_table, edge_table, npart, epart, gb)
    return h, e


# speedup vs baseline: 1.0408x; 1.0408x over previous
import functools

import jax
import jax.numpy as jnp
from jax import lax
from jax.experimental import pallas as pl
from jax.experimental.pallas import tpu as pltpu
from jax.experimental.pallas import tpu_sc as plsc

N_NODES = 10000
N_EDGES = 320000
EDGE_VOCAB = 10000
DIM = 128
L = 16
NV = DIM // L
NC, NS = 2, 16
NW = NC * NS
EPS = 1e-5

NODE_PW = N_NODES // NW
NODE_TAIL = N_NODES - NODE_PW * NW
NODE_CH = 104
NODE_NCH = NODE_PW // NODE_CH
EDGE_PW = N_EDGES // NW
EDGE_CH = 80
EDGE_NCH = EDGE_PW // EDGE_CH
NBUF = 5
DPRE_N = 2

VOC_PW = 624
VOC_TAIL = EDGE_VOCAB - VOC_PW * NS
VOC_CH = 208
VOC_NCH = VOC_PW // VOC_CH
HZERO = 2000

_mesh = plsc.VectorSubcoreMesh(core_axis_name="c", subcore_axis_name="s")

_GATHER_DNUMS = lax.GatherDimensionNumbers(
    offset_dims=(), collapsed_slice_dims=(0,), start_index_map=(0,))


def _bcast(v, i):
    idx = jnp.full((L, 1), i, dtype=jnp.int32)
    return lax.gather(v, idx, _GATHER_DNUMS, (1,),
                      mode=lax.GatherScatterMode.PROMISE_IN_BOUNDS)


def _rsqrt(x):
    i = lax.bitcast_convert_type(x, jnp.int32)
    y = lax.bitcast_convert_type(
        jnp.int32(0x5F3759DF) - lax.shift_right_logical(i, 1), jnp.float32)
    for _ in range(3):
        y = y * (1.5 - 0.5 * x * y * y)
    return y


def _accum_rows(rows_ref, n, carry):
    def body(r, c):
        sums, sqs = c
        new_s, new_q = [], []
        for j in range(NV):
            v = rows_ref[r, pl.ds(L * j, L)]
            new_s.append(sums[j] + v)
            new_q.append(sqs[j] + v * v)
        return tuple(new_s), tuple(new_q)
    return lax.fori_loop(0, n, body, carry)


def _store_carry(part_ref, carry):
    sums, sqs = carry
    for j in range(NV):
        part_ref[0, pl.ds(L * j, L)] = sums[j]
        part_ref[1, pl.ds(L * j, L)] = sqs[j]


def _waccum_rows(rows_ref, cnt_ref, cnt_base, ngroups, carry):
    def body(g, c):
        sums, sqs = c
        cv = cnt_ref[pl.ds(cnt_base + L * g, L)]
        new_s, new_q = list(sums), list(sqs)
        for i in range(L):
            w = _bcast(cv, i)
            for j in range(NV):
                v = rows_ref[L * g + i, pl.ds(L * j, L)]
                wv = w * v
                new_s[j] = new_s[j] + wv
                new_q[j] = new_q[j] + wv * v
        return tuple(new_s), tuple(new_q)
    return lax.fori_loop(0, ngroups, body, carry)


_STATS_SCRATCH = [
    pltpu.VMEM((EDGE_PW,), jnp.int32),
    pltpu.VMEM((NODE_PW + NODE_TAIL,), jnp.int32),
    pltpu.VMEM((NODE_NCH, NODE_CH, DIM), jnp.float32),
    pltpu.VMEM((2, VOC_CH, DIM), jnp.float32),
    pltpu.VMEM((2, DIM), jnp.float32),
    pltpu.VMEM((EDGE_PW,), jnp.float32),
    pltpu.VMEM((HZERO,), jnp.float32),
    pltpu.VMEM((VOC_PW + VOC_TAIL,), jnp.float32),
    pltpu.VMEM_SHARED((EDGE_VOCAB,), jnp.float32),
] + [pltpu.SemaphoreType.DMA] * (NODE_NCH + 3)


@functools.partial(
    pl.kernel,
    out_type=[
        jax.ShapeDtypeStruct((NW, 2, DIM), jnp.float32),
        jax.ShapeDtypeStruct((NW, 2, DIM), jnp.float32),
    ],
    mesh=_mesh,
    scratch_types=_STATS_SCRATCH,
)
def _stats(x_hbm, ea_hbm, ntab_hbm, etab_hbm, npart_hbm, epart_hbm,
           idx_v, nidx_v, nring_v, vrows_v, part_v, ones_v, zero_v, cnt_v,
           hist_sp, *sems):
    nsem = sems[:NODE_NCH]
    ssem = sems[NODE_NCH]
    vsem = sems[NODE_NCH + 1:]
    wid = lax.axis_index("s") * NC + lax.axis_index("c")
    sid = lax.axis_index("s")
    z = jnp.zeros((L,), jnp.float32)
    zero_carry = ((z,) * NV, (z,) * NV)

    nbase = wid * NODE_PW
    pltpu.sync_copy(x_hbm.at[pl.ds(nbase, NODE_PW)], nidx_v.at[pl.ds(0, NODE_PW)])

    @pl.when(wid == NW - 1)
    def _node_tail_idx():
        pltpu.sync_copy(x_hbm.at[pl.ds(NODE_PW * NW, NODE_TAIL)],
                        nidx_v.at[pl.ds(NODE_PW, NODE_TAIL)])

    for k in range(NODE_NCH):
        pltpu.async_copy(
            ntab_hbm.at[nidx_v.at[pl.ds(k * NODE_CH, NODE_CH)]],
            nring_v.at[k], nsem[k])

    ebase = wid * EDGE_PW
    pltpu.sync_copy(ea_hbm.at[pl.ds(ebase, EDGE_PW)], idx_v)

    one16 = jnp.ones((L,), jnp.float32)

    def fill_ones(i, _):
        ones_v[pl.ds(L * i, L)] = one16
        return 0
    lax.fori_loop(0, EDGE_PW // L, fill_ones, 0)

    @pl.when(sid < EDGE_VOCAB // HZERO)
    def _zero_hist():
        def fill_z(i, _):
            zero_v[pl.ds(L * i, L)] = z
            return 0
        lax.fori_loop(0, HZERO // L, fill_z, 0)
        pltpu.sync_copy(zero_v, hist_sp.at[pl.ds(sid * HZERO, HZERO)])

    plsc.subcore_barrier()
    scat = pltpu.async_copy(ones_v, hist_sp.at[idx_v], ssem, add=True)

    carry = zero_carry
    for k in range(NODE_NCH):
        pltpu.make_async_copy(
            ntab_hbm.at[nidx_v.at[pl.ds(0, NODE_CH)]],
            nring_v.at[k], nsem[k]).wait()
        carry = _accum_rows(nring_v.at[k], NODE_CH, carry)
    _store_carry(part_v, carry)

    @pl.when(wid == NW - 1)
    def _node_tail():
        pltpu.async_copy(
            ntab_hbm.at[nidx_v.at[pl.ds(NODE_PW, NODE_TAIL)]],
            nring_v.at[0].at[pl.ds(0, NODE_TAIL)], nsem[0]).wait()
        tsum, tsq = _accum_rows(nring_v.at[0], NODE_TAIL,
                                ((z,) * NV, (z,) * NV))
        for j in range(NV):
            part_v[0, pl.ds(L * j, L)] = part_v[0, pl.ds(L * j, L)] + tsum[j]
            part_v[1, pl.ds(L * j, L)] = part_v[1, pl.ds(L * j, L)] + tsq[j]

    pltpu.sync_copy(part_v, npart_hbm.at[wid])

    scat.wait()
    plsc.subcore_barrier()

    vbase = sid * VOC_PW
    pltpu.sync_copy(hist_sp.at[pl.ds(vbase, VOC_PW)], cnt_v.at[pl.ds(0, VOC_PW)])

    for c in range(2):
        pltpu.async_copy(etab_hbm.at[pl.ds(vbase + c * VOC_CH, VOC_CH)],
                         vrows_v.at[c], vsem[c])
    carry = zero_carry
    for c in range(VOC_NCH):
        b = c % 2
        pltpu.make_async_copy(etab_hbm.at[pl.ds(vbase, VOC_CH)],
                              vrows_v.at[b], vsem[b]).wait()
        carry = _waccum_rows(vrows_v.at[b], cnt_v, c * VOC_CH, VOC_CH // L, carry)
        if c + 2 < VOC_NCH:
            pltpu.async_copy(etab_hbm.at[pl.ds(vbase + (c + 2) * VOC_CH, VOC_CH)],
                             vrows_v.at[b], vsem[b])
    _store_carry(part_v, carry)

    @pl.when(sid == NS - 1)
    def _voc_tail():
        tbase = VOC_PW * NS
        pltpu.sync_copy(hist_sp.at[pl.ds(tbase, VOC_TAIL)],
                        cnt_v.at[pl.ds(VOC_PW, VOC_TAIL)])
        pltpu.sync_copy(etab_hbm.at[pl.ds(tbase, VOC_TAIL)],
                        vrows_v.at[0].at[pl.ds(0, VOC_TAIL)])
        tsum, tsq = _waccum_rows(vrows_v.at[0], cnt_v, VOC_PW, VOC_TAIL // L,
                                 ((z,) * NV, (z,) * NV))
        for j in range(NV):
            part_v[0, pl.ds(L * j, L)] = part_v[0, pl.ds(L * j, L)] + tsum[j]
            part_v[1, pl.ds(L * j, L)] = part_v[1, pl.ds(L * j, L)] + tsq[j]

    pltpu.sync_copy(part_v, epart_hbm.at[wid])


_NORM_SCRATCH = [
    pltpu.VMEM((EDGE_PW,), jnp.int32),
    pltpu.VMEM((NODE_CH, DIM), jnp.float32),
    pltpu.VMEM((NBUF, EDGE_CH, DIM), jnp.float32),
    pltpu.VMEM((4, DIM), jnp.float32),
    pltpu.SemaphoreType.DMA,
] + [pltpu.SemaphoreType.DMA] * NBUF \
  + [pltpu.SemaphoreType.DMA] * NBUF


@functools.partial(
    pl.kernel,
    out_type=[
        jax.ShapeDtypeStruct((N_NODES, DIM), jnp.float32),
        jax.ShapeDtypeStruct((N_EDGES, DIM), jnp.float32),
    ],
    mesh=_mesh,
    scratch_types=_NORM_SCRATCH,
)
def _normalize(x_hbm, ea_hbm, ntab_hbm, etab_hbm, ab_hbm,
               h_hbm, e_hbm, idx_v, nrows_v, ring_v, ab_v,
               nsem, *sems):
    gsem, wsem = sems[:NBUF], sems[NBUF:]
    wid = lax.axis_index("s") * NC + lax.axis_index("c")

    def transform(rows_ref, n, a, b):
        def body(r, _):
            for j in range(NV):
                rows_ref[r, pl.ds(L * j, L)] = (
                    rows_ref[r, pl.ds(L * j, L)] * a[j] + b[j])
            return 0
        lax.fori_loop(0, n, body, 0)

    nbase = wid * NODE_PW
    pltpu.sync_copy(x_hbm.at[pl.ds(nbase, NODE_PW)], idx_v.at[pl.ds(0, NODE_PW)])
    pltpu.sync_copy(ab_hbm, ab_v)
    na = [ab_v[0, pl.ds(L * j, L)] for j in range(NV)]
    nb = [ab_v[1, pl.ds(L * j, L)] for j in range(NV)]
    for k in range(NODE_NCH):
        pltpu.async_copy(
            ntab_hbm.at[idx_v.at[pl.ds(k * NODE_CH, NODE_CH)]],
            nrows_v, nsem).wait()
        transform(nrows_v, NODE_CH, na, nb)
        pltpu.sync_copy(nrows_v,
                        h_hbm.at[pl.ds(nbase + k * NODE_CH, NODE_CH)])

    @pl.when(wid == NW - 1)
    def _node_tail():
        pltpu.sync_copy(x_hbm.at[pl.ds(NODE_PW * NW, NODE_TAIL)],
                        idx_v.at[pl.ds(0, NODE_TAIL)])
        pltpu.async_copy(
            ntab_hbm.at[idx_v.at[pl.ds(0, NODE_TAIL)]],
            nrows_v.at[pl.ds(0, NODE_TAIL)], nsem).wait()
        transform(nrows_v, NODE_TAIL, na, nb)
        pltpu.sync_copy(nrows_v.at[pl.ds(0, NODE_TAIL)],
                        h_hbm.at[pl.ds(NODE_PW * NW, NODE_TAIL)])

    ebase = wid * EDGE_PW
    pltpu.sync_copy(ea_hbm.at[pl.ds(ebase, EDGE_PW)], idx_v)
    ea = [ab_v[2, pl.ds(L * j, L)] for j in range(NV)]
    eb = [ab_v[3, pl.ds(L * j, L)] for j in range(NV)]

    def gather(chunk, buf, sem):
        pltpu.async_copy(
            etab_hbm.at[idx_v.at[pl.ds(chunk * EDGE_CH, EDGE_CH)]],
            ring_v.at[buf], sem)

    def gwait(buf, sem):
        pltpu.make_async_copy(
            etab_hbm.at[idx_v.at[pl.ds(0, EDGE_CH)]],
            ring_v.at[buf], sem).wait()

    def wdrain(buf, sem):
        pltpu.make_async_copy(
            ring_v.at[buf], e_hbm.at[pl.ds(ebase, EDGE_CH)], sem).wait()

    for b in range(DPRE_N):
        gather(b, b, gsem[b])

    def group(g, _):
        for b in range(NBUF):
            k = g * NBUF + b
            gwait(b, gsem[b])
            transform(ring_v.at[b], EDGE_CH, ea, eb)
            pltpu.async_copy(
                ring_v.at[b],
                e_hbm.at[pl.ds(ebase + k * EDGE_CH, EDGE_CH)], wsem[b])
            bn = (b + DPRE_N) % NBUF

            @pl.when(k + DPRE_N < EDGE_NCH)
            def _rearm():
                @pl.when(k + DPRE_N >= NBUF)
                def _drain_prev():
                    wdrain(bn, wsem[bn])
                gather(k + DPRE_N, bn, gsem[bn])
        return 0

    lax.fori_loop(0, EDGE_NCH // NBUF, group, 0)
    for b in range(NBUF):
        wdrain(b, wsem[b])


def _fold_bn(part, n, gamma, beta):
    s = part[:, 0, :].sum(axis=0)
    q = part[:, 1, :].sum(axis=0)
    mean = s / n
    var = q / n - mean * mean
    a = gamma * lax.rsqrt(var + EPS)
    b = beta - mean * a
    return jnp.stack([a, b])


def kernel(x, edge_index, edge_attr, node_table, edge_table,
           node_gamma, node_beta, edge_gamma, edge_beta):
    del edge_index
    x = x.astype(jnp.int32)
    edge_attr = edge_attr.astype(jnp.int32)
    npart, epart = _stats(x, edge_attr, node_table, edge_table)
    ab = jnp.concatenate([
        _fold_bn(npart, N_NODES, node_gamma, node_beta),
        _fold_bn(epart, N_EDGES, edge_gamma, edge_beta)])
    h, e = _normalize(x, edge_attr, node_table, edge_table, ab)
    return h, e


# speedup vs baseline: 1.1030x; 1.0597x over previous
import functools

import jax
import jax.numpy as jnp
from jax import lax
from jax.experimental import pallas as pl
from jax.experimental.pallas import tpu as pltpu
from jax.experimental.pallas import tpu_sc as plsc

N_NODES = 10000
N_EDGES = 320000
EDGE_VOCAB = 10000
DIM = 128
L = 16
NV = DIM // L
NC, NS = 2, 16
NW = NC * NS
EPS = 1e-5

NODE_PW = N_NODES // NW
NODE_TAIL = N_NODES - NODE_PW * NW
NODE_CH = 104
NODE_NCH = NODE_PW // NODE_CH
EDGE_PW = N_EDGES // NW
EDGE_CH = 80
EDGE_NCH = EDGE_PW // EDGE_CH
NBUF = 5
DPRE_N = 2

NCH2 = 200
NRING = 48
NBUF2 = 4
NTAIL2 = (EDGE_PW - NRING * NCH2) // NCH2

VOC_PW = 624
VOC_TAIL = EDGE_VOCAB - VOC_PW * NS
VOC_CH = 208
VOC_NCH = VOC_PW // VOC_CH
HZERO = 2000

_mesh = plsc.VectorSubcoreMesh(core_axis_name="c", subcore_axis_name="s")

_GATHER_DNUMS = lax.GatherDimensionNumbers(
    offset_dims=(), collapsed_slice_dims=(0,), start_index_map=(0,))


def _bcast(v, i):
    idx = jnp.full((L, 1), i, dtype=jnp.int32)
    return lax.gather(v, idx, _GATHER_DNUMS, (1,),
                      mode=lax.GatherScatterMode.PROMISE_IN_BOUNDS)


def _rsqrt(x):
    i = lax.bitcast_convert_type(x, jnp.int32)
    y = lax.bitcast_convert_type(
        jnp.int32(0x5F3759DF) - lax.shift_right_logical(i, 1), jnp.float32)
    for _ in range(3):
        y = y * (1.5 - 0.5 * x * y * y)
    return y


def _accum_rows(rows_ref, n, carry):
    def body(r, c):
        sums, sqs = c
        new_s, new_q = [], []
        for j in range(NV):
            v = rows_ref[r, pl.ds(L * j, L)]
            new_s.append(sums[j] + v)
            new_q.append(sqs[j] + v * v)
        return tuple(new_s), tuple(new_q)
    return lax.fori_loop(0, n, body, carry)


def _store_carry(part_ref, carry):
    sums, sqs = carry
    for j in range(NV):
        part_ref[0, pl.ds(L * j, L)] = sums[j]
        part_ref[1, pl.ds(L * j, L)] = sqs[j]


def _waccum_rows(rows_ref, cnt_ref, cnt_base, ngroups, carry):
    def body(g, c):
        sums, sqs = c
        cv = cnt_ref[pl.ds(cnt_base + L * g, L)]
        new_s, new_q = list(sums), list(sqs)
        for i in range(L):
            w = _bcast(cv, i)
            for j in range(NV):
                v = rows_ref[L * g + i, pl.ds(L * j, L)]
                wv = w * v
                new_s[j] = new_s[j] + wv
                new_q[j] = new_q[j] + wv * v
        return tuple(new_s), tuple(new_q)
    return lax.fori_loop(0, ngroups, body, carry)


_STATS_SCRATCH = [
    pltpu.VMEM((EDGE_PW,), jnp.int32),
    pltpu.VMEM((NODE_PW + NODE_TAIL,), jnp.int32),
    pltpu.VMEM((NODE_NCH, NODE_CH, DIM), jnp.float32),
    pltpu.VMEM((2, VOC_CH, DIM), jnp.float32),
    pltpu.VMEM((2, DIM), jnp.float32),
    pltpu.VMEM((EDGE_PW,), jnp.float32),
    pltpu.VMEM((HZERO,), jnp.float32),
    pltpu.VMEM((VOC_PW + VOC_TAIL,), jnp.float32),
    pltpu.VMEM_SHARED((EDGE_VOCAB,), jnp.float32),
] + [pltpu.SemaphoreType.DMA] * (NODE_NCH + 3)


@functools.partial(
    pl.kernel,
    out_type=[
        jax.ShapeDtypeStruct((NW, 2, DIM), jnp.float32),
        jax.ShapeDtypeStruct((NW, 2, DIM), jnp.float32),
    ],
    mesh=_mesh,
    scratch_types=_STATS_SCRATCH,
)
def _stats(x_hbm, ea_hbm, ntab_hbm, etab_hbm, npart_hbm, epart_hbm,
           idx_v, nidx_v, nring_v, vrows_v, part_v, ones_v, zero_v, cnt_v,
           hist_sp, *sems):
    nsem = sems[:NODE_NCH]
    ssem = sems[NODE_NCH]
    vsem = sems[NODE_NCH + 1:]
    wid = lax.axis_index("s") * NC + lax.axis_index("c")
    sid = lax.axis_index("s")
    z = jnp.zeros((L,), jnp.float32)
    zero_carry = ((z,) * NV, (z,) * NV)

    nbase = wid * NODE_PW
    pltpu.sync_copy(x_hbm.at[pl.ds(nbase, NODE_PW)], nidx_v.at[pl.ds(0, NODE_PW)])

    @pl.when(wid == NW - 1)
    def _node_tail_idx():
        pltpu.sync_copy(x_hbm.at[pl.ds(NODE_PW * NW, NODE_TAIL)],
                        nidx_v.at[pl.ds(NODE_PW, NODE_TAIL)])

    for k in range(NODE_NCH):
        pltpu.async_copy(
            ntab_hbm.at[nidx_v.at[pl.ds(k * NODE_CH, NODE_CH)]],
            nring_v.at[k], nsem[k])

    ebase = wid * EDGE_PW
    pltpu.sync_copy(ea_hbm.at[pl.ds(ebase, EDGE_PW)], idx_v)

    one16 = jnp.ones((L,), jnp.float32)

    def fill_ones(i, _):
        ones_v[pl.ds(L * i, L)] = one16
        return 0
    lax.fori_loop(0, EDGE_PW // L, fill_ones, 0)

    @pl.when(sid < EDGE_VOCAB // HZERO)
    def _zero_hist():
        def fill_z(i, _):
            zero_v[pl.ds(L * i, L)] = z
            return 0
        lax.fori_loop(0, HZERO // L, fill_z, 0)
        pltpu.sync_copy(zero_v, hist_sp.at[pl.ds(sid * HZERO, HZERO)])

    plsc.subcore_barrier()
    scat = pltpu.async_copy(ones_v, hist_sp.at[idx_v], ssem, add=True)

    carry = zero_carry
    for k in range(NODE_NCH):
        pltpu.make_async_copy(
            ntab_hbm.at[nidx_v.at[pl.ds(0, NODE_CH)]],
            nring_v.at[k], nsem[k]).wait()
        carry = _accum_rows(nring_v.at[k], NODE_CH, carry)
    _store_carry(part_v, carry)

    @pl.when(wid == NW - 1)
    def _node_tail():
        pltpu.async_copy(
            ntab_hbm.at[nidx_v.at[pl.ds(NODE_PW, NODE_TAIL)]],
            nring_v.at[0].at[pl.ds(0, NODE_TAIL)], nsem[0]).wait()
        tsum, tsq = _accum_rows(nring_v.at[0], NODE_TAIL,
                                ((z,) * NV, (z,) * NV))
        for j in range(NV):
            part_v[0, pl.ds(L * j, L)] = part_v[0, pl.ds(L * j, L)] + tsum[j]
            part_v[1, pl.ds(L * j, L)] = part_v[1, pl.ds(L * j, L)] + tsq[j]

    pltpu.sync_copy(part_v, npart_hbm.at[wid])

    scat.wait()
    plsc.subcore_barrier()

    vbase = sid * VOC_PW
    pltpu.sync_copy(hist_sp.at[pl.ds(vbase, VOC_PW)], cnt_v.at[pl.ds(0, VOC_PW)])

    for c in range(2):
        pltpu.async_copy(etab_hbm.at[pl.ds(vbase + c * VOC_CH, VOC_CH)],
                         vrows_v.at[c], vsem[c])
    carry = zero_carry
    for c in range(VOC_NCH):
        b = c % 2
        pltpu.make_async_copy(etab_hbm.at[pl.ds(vbase, VOC_CH)],
                              vrows_v.at[b], vsem[b]).wait()
        carry = _waccum_rows(vrows_v.at[b], cnt_v, c * VOC_CH, VOC_CH // L, carry)
        if c + 2 < VOC_NCH:
            pltpu.async_copy(etab_hbm.at[pl.ds(vbase + (c + 2) * VOC_CH, VOC_CH)],
                             vrows_v.at[b], vsem[b])
    _store_carry(part_v, carry)

    @pl.when(sid == NS - 1)
    def _voc_tail():
        tbase = VOC_PW * NS
        pltpu.sync_copy(hist_sp.at[pl.ds(tbase, VOC_TAIL)],
                        cnt_v.at[pl.ds(VOC_PW, VOC_TAIL)])
        pltpu.sync_copy(etab_hbm.at[pl.ds(tbase, VOC_TAIL)],
                        vrows_v.at[0].at[pl.ds(0, VOC_TAIL)])
        tsum, tsq = _waccum_rows(vrows_v.at[0], cnt_v, VOC_PW, VOC_TAIL // L,
                                 ((z,) * NV, (z,) * NV))
        for j in range(NV):
            part_v[0, pl.ds(L * j, L)] = part_v[0, pl.ds(L * j, L)] + tsum[j]
            part_v[1, pl.ds(L * j, L)] = part_v[1, pl.ds(L * j, L)] + tsq[j]

    pltpu.sync_copy(part_v, epart_hbm.at[wid])


_NORM_SCRATCH = [
    pltpu.VMEM((EDGE_PW,), jnp.int32),
    pltpu.VMEM((NODE_CH, DIM), jnp.float32),
    pltpu.VMEM((NBUF2, NCH2, DIM), jnp.float32),
    pltpu.VMEM((4, DIM), jnp.float32),
    pltpu.SemaphoreType.DMA,
] + [pltpu.SemaphoreType.DMA] * NBUF2 \
  + [pltpu.SemaphoreType.DMA] * NBUF2


@functools.partial(
    pl.kernel,
    out_type=[
        jax.ShapeDtypeStruct((N_NODES, DIM), jnp.float32),
        jax.ShapeDtypeStruct((N_EDGES, DIM), jnp.float32),
    ],
    mesh=_mesh,
    scratch_types=_NORM_SCRATCH,
)
def _normalize(x_hbm, ea_hbm, ntab_hbm, etab_hbm, ab_hbm,
               h_hbm, e_hbm, idx_v, nrows_v, ring_v, ab_v,
               nsem, *sems):
    gsem, wsem = sems[:NBUF2], sems[NBUF2:]
    wid = lax.axis_index("s") * NC + lax.axis_index("c")

    def transform(rows_ref, n, a, b):
        def body(r, _):
            for j in range(NV):
                rows_ref[r, pl.ds(L * j, L)] = (
                    rows_ref[r, pl.ds(L * j, L)] * a[j] + b[j])
            return 0
        lax.fori_loop(0, n, body, 0)

    nbase = wid * NODE_PW
    pltpu.sync_copy(x_hbm.at[pl.ds(nbase, NODE_PW)], idx_v.at[pl.ds(0, NODE_PW)])
    pltpu.sync_copy(ab_hbm, ab_v)
    na = [ab_v[0, pl.ds(L * j, L)] for j in range(NV)]
    nb = [ab_v[1, pl.ds(L * j, L)] for j in range(NV)]
    for k in range(NODE_NCH):
        pltpu.async_copy(
            ntab_hbm.at[idx_v.at[pl.ds(k * NODE_CH, NODE_CH)]],
            nrows_v, nsem).wait()
        transform(nrows_v, NODE_CH, na, nb)
        pltpu.sync_copy(nrows_v,
                        h_hbm.at[pl.ds(nbase + k * NODE_CH, NODE_CH)])

    @pl.when(wid == NW - 1)
    def _node_tail():
        pltpu.sync_copy(x_hbm.at[pl.ds(NODE_PW * NW, NODE_TAIL)],
                        idx_v.at[pl.ds(0, NODE_TAIL)])
        pltpu.async_copy(
            ntab_hbm.at[idx_v.at[pl.ds(0, NODE_TAIL)]],
            nrows_v.at[pl.ds(0, NODE_TAIL)], nsem).wait()
        transform(nrows_v, NODE_TAIL, na, nb)
        pltpu.sync_copy(nrows_v.at[pl.ds(0, NODE_TAIL)],
                        h_hbm.at[pl.ds(NODE_PW * NW, NODE_TAIL)])

    ebase = wid * EDGE_PW
    pltpu.sync_copy(ea_hbm.at[pl.ds(ebase, EDGE_PW)], idx_v)
    ea = [ab_v[2, pl.ds(L * j, L)] for j in range(NV)]
    eb = [ab_v[3, pl.ds(L * j, L)] for j in range(NV)]

    def gather(chunk, buf, sem):
        pltpu.async_copy(
            etab_hbm.at[idx_v.at[pl.ds(chunk * NCH2, NCH2)]],
            ring_v.at[buf], sem)

    def gwait(buf, sem):
        pltpu.make_async_copy(
            etab_hbm.at[idx_v.at[pl.ds(0, NCH2)]],
            ring_v.at[buf], sem).wait()

    def wdrain(buf, sem):
        pltpu.make_async_copy(
            ring_v.at[buf], e_hbm.at[pl.ds(ebase, NCH2)], sem).wait()

    for b in range(DPRE_N):
        gather(b, b, gsem[b])

    def group(g, _):
        for b in range(NBUF2):
            k = g * NBUF2 + b
            gwait(b, gsem[b])
            transform(ring_v.at[b], NCH2, ea, eb)
            pltpu.async_copy(
                ring_v.at[b],
                e_hbm.at[pl.ds(ebase + k * NCH2, NCH2)], wsem[b])
            bn = (b + DPRE_N) % NBUF2

            @pl.when(k + DPRE_N < NRING)
            def _rearm():
                @pl.when(k + DPRE_N >= NBUF2)
                def _drain_prev():
                    wdrain(bn, wsem[bn])
                gather(k + DPRE_N, bn, gsem[bn])
        return 0

    lax.fori_loop(0, NRING // NBUF2, group, 0)
    for b in range(NBUF2):
        wdrain(b, wsem[b])
    for t in range(NTAIL2):
        k = NRING + t
        gather(k, 0, gsem[0])
        gwait(0, gsem[0])
        transform(ring_v.at[0], NCH2, ea, eb)
        pltpu.sync_copy(ring_v.at[0],
                        e_hbm.at[pl.ds(ebase + k * NCH2, NCH2)])


def _fold_bn(part, n, gamma, beta):
    s = part[:, 0, :].sum(axis=0)
    q = part[:, 1, :].sum(axis=0)
    mean = s / n
    var = q / n - mean * mean
    a = gamma * lax.rsqrt(var + EPS)
    b = beta - mean * a
    return jnp.stack([a, b])


def kernel(x, edge_index, edge_attr, node_table, edge_table,
           node_gamma, node_beta, edge_gamma, edge_beta):
    del edge_index
    x = x.astype(jnp.int32)
    edge_attr = edge_attr.astype(jnp.int32)
    npart, epart = _stats(x, edge_attr, node_table, edge_table)
    ab = jnp.concatenate([
        _fold_bn(npart, N_NODES, node_gamma, node_beta),
        _fold_bn(epart, N_EDGES, edge_gamma, edge_beta)])
    h, e = _normalize(x, edge_attr, node_table, edge_table, ab)
    return h, e


# speedup vs baseline: 1.1084x; 1.0049x over previous
import functools

import jax
import jax.numpy as jnp
from jax import lax
from jax.experimental import pallas as pl
from jax.experimental.pallas import tpu as pltpu
from jax.experimental.pallas import tpu_sc as plsc

N_NODES = 10000
N_EDGES = 320000
EDGE_VOCAB = 10000
DIM = 128
L = 16
NV = DIM // L
NC, NS = 2, 16
NW = NC * NS
EPS = 1e-5

NODE_PW = N_NODES // NW
NODE_TAIL = N_NODES - NODE_PW * NW
NODE_CH = 104
NODE_NCH = NODE_PW // NODE_CH
EDGE_PW = N_EDGES // NW
EDGE_CH = 80
EDGE_NCH = EDGE_PW // EDGE_CH
NBUF = 5
DPRE_N = 2

NCH2 = 200
NRING = 48
NBUF2 = 4
NTAIL2 = (EDGE_PW - NRING * NCH2) // NCH2

VOC_PW = 624
VOC_TAIL = EDGE_VOCAB - VOC_PW * NS
VOC_CH = 208
VOC_NCH = VOC_PW // VOC_CH
HZERO = 2000

_mesh = plsc.VectorSubcoreMesh(core_axis_name="c", subcore_axis_name="s")

_GATHER_DNUMS = lax.GatherDimensionNumbers(
    offset_dims=(), collapsed_slice_dims=(0,), start_index_map=(0,))


def _bcast(v, i):
    idx = jnp.full((L, 1), i, dtype=jnp.int32)
    return lax.gather(v, idx, _GATHER_DNUMS, (1,),
                      mode=lax.GatherScatterMode.PROMISE_IN_BOUNDS)


def _rsqrt(x):
    i = lax.bitcast_convert_type(x, jnp.int32)
    y = lax.bitcast_convert_type(
        jnp.int32(0x5F3759DF) - lax.shift_right_logical(i, 1), jnp.float32)
    for _ in range(3):
        y = y * (1.5 - 0.5 * x * y * y)
    return y


def _accum_rows(rows_ref, n, carry):
    def body(r, c):
        sums, sqs = c
        new_s, new_q = [], []
        for j in range(NV):
            v = rows_ref[r, pl.ds(L * j, L)]
            new_s.append(sums[j] + v)
            new_q.append(sqs[j] + v * v)
        return tuple(new_s), tuple(new_q)
    return lax.fori_loop(0, n, body, carry)


def _store_carry(part_ref, carry):
    sums, sqs = carry
    for j in range(NV):
        part_ref[0, pl.ds(L * j, L)] = sums[j]
        part_ref[1, pl.ds(L * j, L)] = sqs[j]


def _waccum_rows(rows_ref, cnt_ref, cnt_base, ngroups, carry):
    def body(g, c):
        sums, sqs = c
        cv = cnt_ref[pl.ds(cnt_base + L * g, L)]
        new_s, new_q = list(sums), list(sqs)
        for i in range(L):
            w = _bcast(cv, i)
            for j in range(NV):
                v = rows_ref[L * g + i, pl.ds(L * j, L)]
                wv = w * v
                new_s[j] = new_s[j] + wv
                new_q[j] = new_q[j] + wv * v
        return tuple(new_s), tuple(new_q)
    return lax.fori_loop(0, ngroups, body, carry)


_STATS_SCRATCH = [
    pltpu.VMEM((EDGE_PW,), jnp.int32),
    pltpu.VMEM((NODE_PW + NODE_TAIL,), jnp.int32),
    pltpu.VMEM((NODE_NCH, NODE_CH, DIM), jnp.float32),
    pltpu.VMEM((2, VOC_CH, DIM), jnp.float32),
    pltpu.VMEM((2, DIM), jnp.float32),
    pltpu.VMEM((EDGE_PW,), jnp.float32),
    pltpu.VMEM((HZERO,), jnp.float32),
    pltpu.VMEM((VOC_PW + VOC_TAIL,), jnp.float32),
    pltpu.VMEM_SHARED((EDGE_VOCAB,), jnp.float32),
] + [pltpu.SemaphoreType.DMA] * (NODE_NCH + 3)


@functools.partial(
    pl.kernel,
    out_type=[
        jax.ShapeDtypeStruct((NW, 2, DIM), jnp.float32),
        jax.ShapeDtypeStruct((NW, 2, DIM), jnp.float32),
    ],
    mesh=_mesh,
    scratch_types=_STATS_SCRATCH,
)
def _stats(x_hbm, ea_hbm, ntab_hbm, etab_hbm, npart_hbm, epart_hbm,
           idx_v, nidx_v, nring_v, vrows_v, part_v, ones_v, zero_v, cnt_v,
           hist_sp, *sems):
    nsem = sems[:NODE_NCH]
    ssem = sems[NODE_NCH]
    vsem = sems[NODE_NCH + 1:]
    wid = lax.axis_index("s") * NC + lax.axis_index("c")
    sid = lax.axis_index("s")
    z = jnp.zeros((L,), jnp.float32)
    zero_carry = ((z,) * NV, (z,) * NV)

    nbase = wid * NODE_PW
    pltpu.sync_copy(x_hbm.at[pl.ds(nbase, NODE_PW)], nidx_v.at[pl.ds(0, NODE_PW)])

    @pl.when(wid == NW - 1)
    def _node_tail_idx():
        pltpu.sync_copy(x_hbm.at[pl.ds(NODE_PW * NW, NODE_TAIL)],
                        nidx_v.at[pl.ds(NODE_PW, NODE_TAIL)])

    for k in range(NODE_NCH):
        pltpu.async_copy(
            ntab_hbm.at[nidx_v.at[pl.ds(k * NODE_CH, NODE_CH)]],
            nring_v.at[k], nsem[k])

    ebase = wid * EDGE_PW
    pltpu.sync_copy(ea_hbm.at[pl.ds(ebase, EDGE_PW)], idx_v)

    one16 = jnp.ones((L,), jnp.float32)

    def fill_ones(i, _):
        ones_v[pl.ds(L * i, L)] = one16
        return 0
    lax.fori_loop(0, EDGE_PW // L, fill_ones, 0)

    @pl.when(sid < EDGE_VOCAB // HZERO)
    def _zero_hist():
        def fill_z(i, _):
            zero_v[pl.ds(L * i, L)] = z
            return 0
        lax.fori_loop(0, HZERO // L, fill_z, 0)
        pltpu.sync_copy(zero_v, hist_sp.at[pl.ds(sid * HZERO, HZERO)])

    plsc.subcore_barrier()
    scat = pltpu.async_copy(ones_v, hist_sp.at[idx_v], ssem, add=True)

    carry = zero_carry
    for k in range(NODE_NCH):
        pltpu.make_async_copy(
            ntab_hbm.at[nidx_v.at[pl.ds(0, NODE_CH)]],
            nring_v.at[k], nsem[k]).wait()
        carry = _accum_rows(nring_v.at[k], NODE_CH, carry)
    _store_carry(part_v, carry)

    @pl.when(wid == NW - 1)
    def _node_tail():
        pltpu.async_copy(
            ntab_hbm.at[nidx_v.at[pl.ds(NODE_PW, NODE_TAIL)]],
            nring_v.at[0].at[pl.ds(0, NODE_TAIL)], nsem[0]).wait()
        tsum, tsq = _accum_rows(nring_v.at[0], NODE_TAIL,
                                ((z,) * NV, (z,) * NV))
        for j in range(NV):
            part_v[0, pl.ds(L * j, L)] = part_v[0, pl.ds(L * j, L)] + tsum[j]
            part_v[1, pl.ds(L * j, L)] = part_v[1, pl.ds(L * j, L)] + tsq[j]

    pltpu.sync_copy(part_v, npart_hbm.at[wid])

    scat.wait()
    plsc.subcore_barrier()

    vbase = sid * VOC_PW
    pltpu.sync_copy(hist_sp.at[pl.ds(vbase, VOC_PW)], cnt_v.at[pl.ds(0, VOC_PW)])

    for c in range(2):
        pltpu.async_copy(etab_hbm.at[pl.ds(vbase + c * VOC_CH, VOC_CH)],
                         vrows_v.at[c], vsem[c])
    carry = zero_carry
    for c in range(VOC_NCH):
        b = c % 2
        pltpu.make_async_copy(etab_hbm.at[pl.ds(vbase, VOC_CH)],
                              vrows_v.at[b], vsem[b]).wait()
        carry = _waccum_rows(vrows_v.at[b], cnt_v, c * VOC_CH, VOC_CH // L, carry)
        if c + 2 < VOC_NCH:
            pltpu.async_copy(etab_hbm.at[pl.ds(vbase + (c + 2) * VOC_CH, VOC_CH)],
                             vrows_v.at[b], vsem[b])
    _store_carry(part_v, carry)

    @pl.when(sid == NS - 1)
    def _voc_tail():
        tbase = VOC_PW * NS
        pltpu.sync_copy(hist_sp.at[pl.ds(tbase, VOC_TAIL)],
                        cnt_v.at[pl.ds(VOC_PW, VOC_TAIL)])
        pltpu.sync_copy(etab_hbm.at[pl.ds(tbase, VOC_TAIL)],
                        vrows_v.at[0].at[pl.ds(0, VOC_TAIL)])
        tsum, tsq = _waccum_rows(vrows_v.at[0], cnt_v, VOC_PW, VOC_TAIL // L,
                                 ((z,) * NV, (z,) * NV))
        for j in range(NV):
            part_v[0, pl.ds(L * j, L)] = part_v[0, pl.ds(L * j, L)] + tsum[j]
            part_v[1, pl.ds(L * j, L)] = part_v[1, pl.ds(L * j, L)] + tsq[j]

    pltpu.sync_copy(part_v, epart_hbm.at[wid])


_NORM_SCRATCH = [
    pltpu.VMEM((EDGE_PW,), jnp.int32),
    pltpu.VMEM((NODE_CH, DIM), jnp.float32),
    pltpu.VMEM((NBUF2, NCH2, DIM), jnp.float32),
    pltpu.VMEM((4, DIM), jnp.float32),
    pltpu.SemaphoreType.DMA,
] + [pltpu.SemaphoreType.DMA] * NBUF2 \
  + [pltpu.SemaphoreType.DMA] * NBUF2


@functools.partial(
    pl.kernel,
    out_type=[
        jax.ShapeDtypeStruct((N_NODES, DIM), jnp.float32),
        jax.ShapeDtypeStruct((N_EDGES, DIM), jnp.float32),
    ],
    mesh=_mesh,
    scratch_types=_NORM_SCRATCH,
)
def _normalize(x_hbm, ea_hbm, ntab_hbm, etab_hbm, ab_hbm,
               h_hbm, e_hbm, idx_v, nrows_v, ring_v, ab_v,
               nsem, *sems):
    gsem, wsem = sems[:NBUF2], sems[NBUF2:]
    wid = lax.axis_index("s") * NC + lax.axis_index("c")

    def transform(rows_ref, n, a, b):
        def body(r2, _):
            for u in range(2):
                for j in range(NV):
                    rows_ref[2 * r2 + u, pl.ds(L * j, L)] = (
                        rows_ref[2 * r2 + u, pl.ds(L * j, L)] * a[j] + b[j])
            return 0
        lax.fori_loop(0, n // 2, body, 0)

    nbase = wid * NODE_PW
    pltpu.sync_copy(x_hbm.at[pl.ds(nbase, NODE_PW)], idx_v.at[pl.ds(0, NODE_PW)])
    pltpu.sync_copy(ab_hbm, ab_v)
    na = [ab_v[0, pl.ds(L * j, L)] for j in range(NV)]
    nb = [ab_v[1, pl.ds(L * j, L)] for j in range(NV)]
    for k in range(NODE_NCH):
        pltpu.async_copy(
            ntab_hbm.at[idx_v.at[pl.ds(k * NODE_CH, NODE_CH)]],
            nrows_v, nsem).wait()
        transform(nrows_v, NODE_CH, na, nb)
        pltpu.sync_copy(nrows_v,
                        h_hbm.at[pl.ds(nbase + k * NODE_CH, NODE_CH)])

    @pl.when(wid == NW - 1)
    def _node_tail():
        pltpu.sync_copy(x_hbm.at[pl.ds(NODE_PW * NW, NODE_TAIL)],
                        idx_v.at[pl.ds(0, NODE_TAIL)])
        pltpu.async_copy(
            ntab_hbm.at[idx_v.at[pl.ds(0, NODE_TAIL)]],
            nrows_v.at[pl.ds(0, NODE_TAIL)], nsem).wait()
        transform(nrows_v, NODE_TAIL, na, nb)
        pltpu.sync_copy(nrows_v.at[pl.ds(0, NODE_TAIL)],
                        h_hbm.at[pl.ds(NODE_PW * NW, NODE_TAIL)])

    ebase = wid * EDGE_PW
    pltpu.sync_copy(ea_hbm.at[pl.ds(ebase, EDGE_PW)], idx_v)
    ea = [ab_v[2, pl.ds(L * j, L)] for j in range(NV)]
    eb = [ab_v[3, pl.ds(L * j, L)] for j in range(NV)]

    def gather(chunk, buf, sem):
        pltpu.async_copy(
            etab_hbm.at[idx_v.at[pl.ds(chunk * NCH2, NCH2)]],
            ring_v.at[buf], sem)

    def gwait(buf, sem):
        pltpu.make_async_copy(
            etab_hbm.at[idx_v.at[pl.ds(0, NCH2)]],
            ring_v.at[buf], sem).wait()

    def wdrain(buf, sem):
        pltpu.make_async_copy(
            ring_v.at[buf], e_hbm.at[pl.ds(ebase, NCH2)], sem).wait()

    for b in range(DPRE_N):
        gather(b, b, gsem[b])

    def group(g, _):
        for b in range(NBUF2):
            k = g * NBUF2 + b
            gwait(b, gsem[b])
            transform(ring_v.at[b], NCH2, ea, eb)
            pltpu.async_copy(
                ring_v.at[b],
                e_hbm.at[pl.ds(ebase + k * NCH2, NCH2)], wsem[b])
            bn = (b + DPRE_N) % NBUF2

            @pl.when(k + DPRE_N < NRING)
            def _rearm():
                @pl.when(k + DPRE_N >= NBUF2)
                def _drain_prev():
                    wdrain(bn, wsem[bn])
                gather(k + DPRE_N, bn, gsem[bn])
        return 0

    lax.fori_loop(0, NRING // NBUF2, group, 0)
    for b in range(NBUF2):
        wdrain(b, wsem[b])
    for t in range(NTAIL2):
        k = NRING + t
        gather(k, 0, gsem[0])
        gwait(0, gsem[0])
        transform(ring_v.at[0], NCH2, ea, eb)
        pltpu.sync_copy(ring_v.at[0],
                        e_hbm.at[pl.ds(ebase + k * NCH2, NCH2)])


def _fold_bn(part, n, gamma, beta):
    s = part[:, 0, :].sum(axis=0)
    q = part[:, 1, :].sum(axis=0)
    mean = s / n
    var = q / n - mean * mean
    a = gamma * lax.rsqrt(var + EPS)
    b = beta - mean * a
    return jnp.stack([a, b])


def kernel(x, edge_index, edge_attr, node_table, edge_table,
           node_gamma, node_beta, edge_gamma, edge_beta):
    del edge_index
    x = x.astype(jnp.int32)
    edge_attr = edge_attr.astype(jnp.int32)
    npart, epart = _stats(x, edge_attr, node_table, edge_table)
    ab = jnp.concatenate([
        _fold_bn(npart, N_NODES, node_gamma, node_beta),
        _fold_bn(epart, N_EDGES, edge_gamma, edge_beta)])
    h, e = _normalize(x, edge_attr, node_table, edge_table, ab)
    return h, e


# speedup vs baseline: 1.1095x; 1.0011x over previous
import functools

import jax
import jax.numpy as jnp
from jax import lax
from jax.experimental import pallas as pl
from jax.experimental.pallas import tpu as pltpu
from jax.experimental.pallas import tpu_sc as plsc

N_NODES = 10000
N_EDGES = 320000
EDGE_VOCAB = 10000
DIM = 128
L = 16
NV = DIM // L
NC, NS = 2, 16
NW = NC * NS
EPS = 1e-5

NODE_PW = N_NODES // NW
NODE_TAIL = N_NODES - NODE_PW * NW
NODE_CH = 104
NODE_NCH = NODE_PW // NODE_CH
EDGE_PW = N_EDGES // NW
DPRE_N = 2

NCH2 = 200
NRING = 48
NBUF2 = 4
NTAIL2 = (EDGE_PW - NRING * NCH2) // NCH2

VOC_PW = 624
VOC_TAIL = EDGE_VOCAB - VOC_PW * NS
VOC_CH = 208
VOC_NCH = VOC_PW // VOC_CH
HZERO = 2000

_mesh = plsc.VectorSubcoreMesh(core_axis_name="c", subcore_axis_name="s")

_GATHER_DNUMS = lax.GatherDimensionNumbers(
    offset_dims=(), collapsed_slice_dims=(0,), start_index_map=(0,))


def _bcast(v, i):
    idx = jnp.full((L, 1), i, dtype=jnp.int32)
    return lax.gather(v, idx, _GATHER_DNUMS, (1,),
                      mode=lax.GatherScatterMode.PROMISE_IN_BOUNDS)


def _accum_rows(rows_ref, n, carry):
    def body(r, c):
        sums, sqs = c
        new_s, new_q = [], []
        for j in range(NV):
            v = rows_ref[r, pl.ds(L * j, L)]
            new_s.append(sums[j] + v)
            new_q.append(sqs[j] + v * v)
        return tuple(new_s), tuple(new_q)
    return lax.fori_loop(0, n, body, carry)


def _store_carry(part_ref, carry):
    sums, sqs = carry
    for j in range(NV):
        part_ref[0, pl.ds(L * j, L)] = sums[j]
        part_ref[1, pl.ds(L * j, L)] = sqs[j]


def _waccum_rows(rows_ref, cnt_ref, cnt_base, ngroups, carry):
    def body(g, c):
        sums, sqs = c
        cv = cnt_ref[pl.ds(cnt_base + L * g, L)]
        new_s, new_q = list(sums), list(sqs)
        for i in range(L):
            w = _bcast(cv, i)
            for j in range(NV):
                v = rows_ref[L * g + i, pl.ds(L * j, L)]
                wv = w * v
                new_s[j] = new_s[j] + wv
                new_q[j] = new_q[j] + wv * v
        return tuple(new_s), tuple(new_q)
    return lax.fori_loop(0, ngroups, body, carry)


_STATS_SCRATCH = [
    pltpu.VMEM((EDGE_PW,), jnp.int32),
    pltpu.VMEM((NODE_PW + NODE_TAIL,), jnp.int32),
    pltpu.VMEM((NODE_NCH, NODE_CH, DIM), jnp.float32),
    pltpu.VMEM((2, VOC_CH, DIM), jnp.float32),
    pltpu.VMEM((2, DIM), jnp.float32),
    pltpu.VMEM((EDGE_PW,), jnp.float32),
    pltpu.VMEM((HZERO,), jnp.float32),
    pltpu.VMEM((VOC_PW + VOC_TAIL,), jnp.float32),
    pltpu.VMEM_SHARED((EDGE_VOCAB,), jnp.float32),
] + [pltpu.SemaphoreType.DMA] * (NODE_NCH + 3)


@functools.partial(
    pl.kernel,
    out_type=[
        jax.ShapeDtypeStruct((NW, 2, DIM), jnp.float32),
        jax.ShapeDtypeStruct((NW, 2, DIM), jnp.float32),
    ],
    mesh=_mesh,
    scratch_types=_STATS_SCRATCH,
)
def _stats(x_hbm, ea_hbm, ntab_hbm, etab_hbm, npart_hbm, epart_hbm,
           idx_v, nidx_v, nring_v, vrows_v, part_v, ones_v, zero_v, cnt_v,
           hist_sp, *sems):
    nsem = sems[:NODE_NCH]
    ssem = sems[NODE_NCH]
    vsem = sems[NODE_NCH + 1:]
    wid = lax.axis_index("s") * NC + lax.axis_index("c")
    sid = lax.axis_index("s")
    z = jnp.zeros((L,), jnp.float32)
    zero_carry = ((z,) * NV, (z,) * NV)

    nbase = wid * NODE_PW
    pltpu.sync_copy(x_hbm.at[pl.ds(nbase, NODE_PW)], nidx_v.at[pl.ds(0, NODE_PW)])

    @pl.when(wid == NW - 1)
    def _node_tail_idx():
        pltpu.sync_copy(x_hbm.at[pl.ds(NODE_PW * NW, NODE_TAIL)],
                        nidx_v.at[pl.ds(NODE_PW, NODE_TAIL)])

    for k in range(NODE_NCH):
        pltpu.async_copy(
            ntab_hbm.at[nidx_v.at[pl.ds(k * NODE_CH, NODE_CH)]],
            nring_v.at[k], nsem[k])

    ebase = wid * EDGE_PW
    pltpu.sync_copy(ea_hbm.at[pl.ds(ebase, EDGE_PW)], idx_v)

    one16 = jnp.ones((L,), jnp.float32)

    def fill_ones(i, _):
        ones_v[pl.ds(L * i, L)] = one16
        return 0
    lax.fori_loop(0, EDGE_PW // L, fill_ones, 0)

    @pl.when(sid < EDGE_VOCAB // HZERO)
    def _zero_hist():
        def fill_z(i, _):
            zero_v[pl.ds(L * i, L)] = z
            return 0
        lax.fori_loop(0, HZERO // L, fill_z, 0)
        pltpu.sync_copy(zero_v, hist_sp.at[pl.ds(sid * HZERO, HZERO)])

    plsc.subcore_barrier()
    scat = pltpu.async_copy(ones_v, hist_sp.at[idx_v], ssem, add=True)

    carry = zero_carry
    for k in range(NODE_NCH):
        pltpu.make_async_copy(
            ntab_hbm.at[nidx_v.at[pl.ds(0, NODE_CH)]],
            nring_v.at[k], nsem[k]).wait()
        carry = _accum_rows(nring_v.at[k], NODE_CH, carry)
    _store_carry(part_v, carry)

    @pl.when(wid == NW - 1)
    def _node_tail():
        pltpu.async_copy(
            ntab_hbm.at[nidx_v.at[pl.ds(NODE_PW, NODE_TAIL)]],
            nring_v.at[0].at[pl.ds(0, NODE_TAIL)], nsem[0]).wait()
        tsum, tsq = _accum_rows(nring_v.at[0], NODE_TAIL,
                                ((z,) * NV, (z,) * NV))
        for j in range(NV):
            part_v[0, pl.ds(L * j, L)] = part_v[0, pl.ds(L * j, L)] + tsum[j]
            part_v[1, pl.ds(L * j, L)] = part_v[1, pl.ds(L * j, L)] + tsq[j]

    pltpu.sync_copy(part_v, npart_hbm.at[wid])

    scat.wait()
    plsc.subcore_barrier()

    vbase = sid * VOC_PW
    pltpu.sync_copy(hist_sp.at[pl.ds(vbase, VOC_PW)], cnt_v.at[pl.ds(0, VOC_PW)])

    for c in range(2):
        pltpu.async_copy(etab_hbm.at[pl.ds(vbase + c * VOC_CH, VOC_CH)],
                         vrows_v.at[c], vsem[c])
    carry = zero_carry
    for c in range(VOC_NCH):
        b = c % 2
        pltpu.make_async_copy(etab_hbm.at[pl.ds(vbase, VOC_CH)],
                              vrows_v.at[b], vsem[b]).wait()
        carry = _waccum_rows(vrows_v.at[b], cnt_v, c * VOC_CH, VOC_CH // L, carry)
        if c + 2 < VOC_NCH:
            pltpu.async_copy(etab_hbm.at[pl.ds(vbase + (c + 2) * VOC_CH, VOC_CH)],
                             vrows_v.at[b], vsem[b])
    _store_carry(part_v, carry)

    @pl.when(sid == NS - 1)
    def _voc_tail():
        tbase = VOC_PW * NS
        pltpu.sync_copy(hist_sp.at[pl.ds(tbase, VOC_TAIL)],
                        cnt_v.at[pl.ds(VOC_PW, VOC_TAIL)])
        pltpu.sync_copy(etab_hbm.at[pl.ds(tbase, VOC_TAIL)],
                        vrows_v.at[0].at[pl.ds(0, VOC_TAIL)])
        tsum, tsq = _waccum_rows(vrows_v.at[0], cnt_v, VOC_PW, VOC_TAIL // L,
                                 ((z,) * NV, (z,) * NV))
        for j in range(NV):
            part_v[0, pl.ds(L * j, L)] = part_v[0, pl.ds(L * j, L)] + tsum[j]
            part_v[1, pl.ds(L * j, L)] = part_v[1, pl.ds(L * j, L)] + tsq[j]

    pltpu.sync_copy(part_v, epart_hbm.at[wid])


_NORM_SCRATCH = [
    pltpu.VMEM((EDGE_PW,), jnp.int32),
    pltpu.VMEM((NODE_CH, DIM), jnp.float32),
    pltpu.VMEM((NBUF2, NCH2, DIM), jnp.float32),
    pltpu.VMEM((4, DIM), jnp.float32),
    pltpu.SemaphoreType.DMA,
] + [pltpu.SemaphoreType.DMA] * NBUF2 \
  + [pltpu.SemaphoreType.DMA] * NBUF2


@functools.partial(
    pl.kernel,
    out_type=[
        jax.ShapeDtypeStruct((N_NODES, DIM), jnp.float32),
        jax.ShapeDtypeStruct((N_EDGES, DIM), jnp.float32),
    ],
    mesh=_mesh,
    scratch_types=_NORM_SCRATCH,
)
def _normalize(x_hbm, ea_hbm, ntab_hbm, etab_hbm, ab_hbm,
               h_hbm, e_hbm, idx_v, nrows_v, ring_v, ab_v,
               nsem, *sems):
    gsem, wsem = sems[:NBUF2], sems[NBUF2:]
    wid = lax.axis_index("s") * NC + lax.axis_index("c")

    def transform(rows_ref, n, a, b):
        def body(r2, _):
            for u in range(2):
                for j in range(NV):
                    rows_ref[2 * r2 + u, pl.ds(L * j, L)] = (
                        rows_ref[2 * r2 + u, pl.ds(L * j, L)] * a[j] + b[j])
            return 0
        lax.fori_loop(0, n // 2, body, 0)

    nbase = wid * NODE_PW
    pltpu.sync_copy(x_hbm.at[pl.ds(nbase, NODE_PW)], idx_v.at[pl.ds(0, NODE_PW)])
    pltpu.sync_copy(ab_hbm, ab_v)
    na = [ab_v[0, pl.ds(L * j, L)] for j in range(NV)]
    nb = [ab_v[1, pl.ds(L * j, L)] for j in range(NV)]
    for k in range(NODE_NCH):
        pltpu.async_copy(
            ntab_hbm.at[idx_v.at[pl.ds(k * NODE_CH, NODE_CH)]],
            nrows_v, nsem).wait()
        transform(nrows_v, NODE_CH, na, nb)
        pltpu.sync_copy(nrows_v,
                        h_hbm.at[pl.ds(nbase + k * NODE_CH, NODE_CH)])

    @pl.when(wid == NW - 1)
    def _node_tail():
        pltpu.sync_copy(x_hbm.at[pl.ds(NODE_PW * NW, NODE_TAIL)],
                        idx_v.at[pl.ds(0, NODE_TAIL)])
        pltpu.async_copy(
            ntab_hbm.at[idx_v.at[pl.ds(0, NODE_TAIL)]],
            nrows_v.at[pl.ds(0, NODE_TAIL)], nsem).wait()
        transform(nrows_v, NODE_TAIL, na, nb)
        pltpu.sync_copy(nrows_v.at[pl.ds(0, NODE_TAIL)],
                        h_hbm.at[pl.ds(NODE_PW * NW, NODE_TAIL)])

    ebase = wid * EDGE_PW
    pltpu.sync_copy(ea_hbm.at[pl.ds(ebase, EDGE_PW)], idx_v)
    ea = [ab_v[2, pl.ds(L * j, L)] for j in range(NV)]
    eb = [ab_v[3, pl.ds(L * j, L)] for j in range(NV)]

    def gather(chunk, buf, sem):
        pltpu.async_copy(
            etab_hbm.at[idx_v.at[pl.ds(chunk * NCH2, NCH2)]],
            ring_v.at[buf], sem)

    def gwait(buf, sem):
        pltpu.make_async_copy(
            etab_hbm.at[idx_v.at[pl.ds(0, NCH2)]],
            ring_v.at[buf], sem).wait()

    def wdrain(buf, sem):
        pltpu.make_async_copy(
            ring_v.at[buf], e_hbm.at[pl.ds(ebase, NCH2)], sem).wait()

    for b in range(DPRE_N):
        gather(b, b, gsem[b])

    def group(g, _):
        for b in range(NBUF2):
            k = g * NBUF2 + b
            gwait(b, gsem[b])
            transform(ring_v.at[b], NCH2, ea, eb)
            pltpu.async_copy(
                ring_v.at[b],
                e_hbm.at[pl.ds(ebase + k * NCH2, NCH2)], wsem[b])
            bn = (b + DPRE_N) % NBUF2

            @pl.when(k + DPRE_N < NRING)
            def _rearm():
                @pl.when(k + DPRE_N >= NBUF2)
                def _drain_prev():
                    wdrain(bn, wsem[bn])
                gather(k + DPRE_N, bn, gsem[bn])
        return 0

    lax.fori_loop(0, NRING // NBUF2, group, 0)
    for b in range(NBUF2):
        wdrain(b, wsem[b])
    for t in range(NTAIL2):
        k = NRING + t
        gather(k, 0, gsem[0])
        gwait(0, gsem[0])
        transform(ring_v.at[0], NCH2, ea, eb)
        pltpu.sync_copy(ring_v.at[0],
                        e_hbm.at[pl.ds(ebase + k * NCH2, NCH2)])


def _fold_bn(part, n, gamma, beta):
    s = part[:, 0, :].sum(axis=0)
    q = part[:, 1, :].sum(axis=0)
    mean = s / n
    var = q / n - mean * mean
    a = gamma * lax.rsqrt(var + EPS)
    b = beta - mean * a
    return jnp.stack([a, b])


def kernel(x, edge_index, edge_attr, node_table, edge_table,
           node_gamma, node_beta, edge_gamma, edge_beta):
    del edge_index
    x = x.astype(jnp.int32)
    edge_attr = edge_attr.astype(jnp.int32)
    npart, epart = _stats(x, edge_attr, node_table, edge_table)
    ab = jnp.concatenate([
        _fold_bn(npart, N_NODES, node_gamma, node_beta),
        _fold_bn(epart, N_EDGES, edge_gamma, edge_beta)])
    h, e = _normalize(x, edge_attr, node_table, edge_table, ab)
    return h, e
